# baseline (speedup 1.0000x reference)
import functools

import jax
import jax.numpy as jnp
from jax import lax
from jax.experimental import pallas as pl
from jax.experimental.pallas import tpu as pltpu
from jax.experimental.pallas import tpu_sc as plsc

NUM_EXPERT = 8
D_MODEL = 768
D_FF = 4 * D_MODEL
TOP_K = 2
BATCH = 4096
N_TOKENS = BATCH // TOP_K

M = 256
T = BATCH // M + NUM_EXPERT
NSORT = T * M

NC = 2
NS = 16
NW = NC * NS

_G_PER_W = NSORT // NW
_G_CHUNK = _G_PER_W // 4
_C_PER_W = N_TOKENS // NW
_LANES = 16

HALF_T = T // 2
HALF_ROWS = NSORT // 2
_GH_PER_W = HALF_ROWS // NW
_GH_CHUNK = _GH_PER_W // 2


@functools.cache
def _get_sc_gather():
    mesh = plsc.VectorSubcoreMesh(core_axis_name="c", subcore_axis_name="s")
    nbuf = 2
    nch = _GH_PER_W // _GH_CHUNK

    @functools.partial(
        pl.kernel,
        out_type=jax.ShapeDtypeStruct((HALF_ROWS, D_MODEL), jnp.float32),
        mesh=mesh,
        scratch_types=(
            [pltpu.VMEM((nch, _GH_CHUNK), jnp.int32)]
            + [pltpu.VMEM((_GH_CHUNK, D_MODEL), jnp.float32)] * nbuf
            + [pltpu.SemaphoreType.DMA] * (2 * nbuf)
        ),
    )
    def _sc_gather_k(inp_hbm, idx_hbm, out_hbm, idx_v, b0, b1,
                     g0, g1, s0, s1):
        bufs = [b0, b1]
        gsems = [g0, g1]
        ssems = [s0, s1]
        wid = lax.axis_index("s") * NC + lax.axis_index("c")
        base = wid * _GH_PER_W
        pltpu.sync_copy(idx_hbm.at[wid], idx_v)
        hg = [
            pltpu.async_copy(inp_hbm.at[idx_v.at[c]], bufs[c], gsems[c])
            for c in range(nbuf)
        ]
        hs = [None] * nbuf
        for c in range(nch):
            b = c % nbuf
            if c >= nbuf:
                hs[b].wait()
                hg.append(pltpu.async_copy(
                    inp_hbm.at[idx_v.at[c]], bufs[b], gsems[b]))
            hg[c].wait()
            hs[b] = pltpu.async_copy(
                bufs[b], out_hbm.at[pl.ds(base + c * _GH_CHUNK, _GH_CHUNK)],
                ssems[b])
        for c in range(max(0, nch - nbuf), nch):
            b = c % nbuf
            if hs[b] is not None:
                hs[b].wait()
                hs[b] = None

    return _sc_gather_k


def _sc_gather(inp, srow):
    return _get_sc_gather()(inp, srow)


@functools.cache
def _get_sc_combine():
    mesh = plsc.VectorSubcoreMesh(core_axis_name="c", subcore_axis_name="s")

    @functools.partial(
        pl.kernel,
        out_type=jax.ShapeDtypeStruct((N_TOKENS, D_MODEL), jnp.float32),
        mesh=mesh,
        scratch_types=[
            pltpu.VMEM((2 * _C_PER_W,), jnp.int32),
            pltpu.VMEM((2 * _C_PER_W, D_MODEL), jnp.float32),
            pltpu.SemaphoreType.DMA,
        ],
    )
    def _sc_combine_k(y_hbm, pos_hbm, out_hbm, idx_v, buf, sem):
        wid = lax.axis_index("s") * NC + lax.axis_index("c")
        tbase = wid * _C_PER_W
        pltpu.sync_copy(pos_hbm.at[pl.ds(2 * tbase, 2 * _C_PER_W)], idx_v)
        pltpu.async_copy(y_hbm.at[idx_v], buf, sem).wait()

        def body(i, carry):
            for c in range(D_MODEL // _LANES):
                sl = pl.ds(c * _LANES, _LANES)
                buf[i, sl] = buf[2 * i, sl] + buf[2 * i + 1, sl]
            return carry

        lax.fori_loop(0, _C_PER_W, body, 0)
        pltpu.sync_copy(
            buf.at[pl.ds(0, _C_PER_W)], out_hbm.at[pl.ds(tbase, _C_PER_W)])

    return _sc_combine_k


def _sc_combine(y_scaled, pos):
    return _get_sc_combine()(y_scaled, pos)


def _ffn_kernel(eid_ref, flag_ref,
                x_ref, w1_ref, w2_ref, score_ref,
                y_ref):
    t = pl.program_id(0)

    @pl.when(flag_ref[t] == 1)
    def _body():
        h = lax.dot_general(
            x_ref[...], w1_ref[0],
            (((1,), (1,)), ((), ())), preferred_element_type=jnp.float32)
        h = jax.nn.gelu(h, approximate=True)
        y = lax.dot_general(
            h, w2_ref[0],
            (((1,), (1,)), ((), ())), preferred_element_type=jnp.float32)
        y_ref[...] = y * score_ref[...]


def _ffn_kernel_alias(eid_ref, flag_ref, yprev_ref,
                      x_ref, w1_ref, w2_ref, score_ref, y_ref):
    del yprev_ref
    _ffn_kernel(eid_ref, flag_ref, x_ref, w1_ref, w2_ref, score_ref, y_ref)


def _ffn_half0(x_half, tile_eid, tile_flag, score_half, w1, w2):
    grid_spec = pltpu.PrefetchScalarGridSpec(
        num_scalar_prefetch=2,
        grid=(HALF_T,),
        in_specs=[
            pl.BlockSpec((M, D_MODEL), lambda t, e, f: (t, 0)),
            pl.BlockSpec((1, D_FF, D_MODEL), lambda t, e, f: (e[t], 0, 0)),
            pl.BlockSpec((1, D_MODEL, D_FF), lambda t, e, f: (e[t], 0, 0)),
            pl.BlockSpec((M, 1), lambda t, e, f: (t, 0)),
        ],
        out_specs=pl.BlockSpec((M, D_MODEL), lambda t, e, f: (t, 0)),
    )
    return pl.pallas_call(
        _ffn_kernel,
        grid_spec=grid_spec,
        out_shape=jax.ShapeDtypeStruct((NSORT, D_MODEL), jnp.float32),
    )(tile_eid, tile_flag, x_half, w1, w2, score_half)


def _ffn_half1(y_prev, x_half, tile_eid, tile_flag, score_half, w1, w2):
    grid_spec = pltpu.PrefetchScalarGridSpec(
        num_scalar_prefetch=2,
        grid=(HALF_T,),
        in_specs=[
            pl.BlockSpec(memory_space=pl.ANY),
            pl.BlockSpec((M, D_MODEL), lambda t, e, f: (t, 0)),
            pl.BlockSpec((1, D_FF, D_MODEL), lambda t, e, f: (e[t], 0, 0)),
            pl.BlockSpec((1, D_MODEL, D_FF), lambda t, e, f: (e[t], 0, 0)),
            pl.BlockSpec((M, 1), lambda t, e, f: (t, 0)),
        ],
        out_specs=pl.BlockSpec((M, D_MODEL), lambda t, e, f: (HALF_T + t, 0)),
    )
    return pl.pallas_call(
        _ffn_kernel_alias,
        grid_spec=grid_spec,
        out_shape=jax.ShapeDtypeStruct((NSORT, D_MODEL), jnp.float32),
        input_output_aliases={2: 0},
    )(tile_eid, tile_flag, y_prev, x_half, w1, w2, score_half)


def kernel(inp, gate_idx, gate_score, weight_htoh4, weight_h4toh):
    g = gate_idx.astype(jnp.int32)
    onehot = (g[:, None] == jnp.arange(NUM_EXPERT)[None, :]).astype(jnp.int32)
    incl = jnp.cumsum(onehot, axis=0)
    rank = jnp.sum((incl - onehot) * onehot, axis=1)
    counts = incl[-1]
    tiles_e = (counts + M - 1) // M
    tstart = jnp.concatenate(
        [jnp.zeros((1,), jnp.int32), jnp.cumsum(tiles_e)[:-1].astype(jnp.int32)])

    t = jnp.arange(T, dtype=jnp.int32)
    belongs = (t[:, None] >= tstart[None, :]) & (
        t[:, None] < (tstart + tiles_e)[None, :])
    has_e = belongs.any(axis=1)
    raw_eid = jnp.where(has_e, jnp.argmax(belongs, axis=1), 0).astype(jnp.int32)
    tile_eid = lax.cummax(raw_eid)
    tile_flag = has_e.astype(jnp.int32)

    pos = (tstart[g] * M + rank).astype(jnp.int32)
    pad_rows = jnp.arange(NSORT, dtype=jnp.int32) % BATCH
    srow = pad_rows.at[pos].set(jnp.arange(BATCH, dtype=jnp.int32))
    score_sorted = jnp.zeros((NSORT,), jnp.float32).at[pos].set(
        gate_score.reshape(-1)).reshape(NSORT, 1)

    srow3 = srow.reshape(2, NW, -1, _GH_CHUNK)
    x0 = _sc_gather(inp, srow3[0])
    x1 = _sc_gather(inp, srow3[1])
    y0 = _ffn_half0(x0, tile_eid[:HALF_T], tile_flag[:HALF_T],
                    score_sorted[:HALF_ROWS], weight_htoh4, weight_h4toh)
    y = _ffn_half1(y0, x1, tile_eid[HALF_T:], tile_flag[HALF_T:],
                   score_sorted[HALF_ROWS:], weight_htoh4, weight_h4toh)
    return _sc_combine(y, pos)


# speedup vs baseline: 1.0549x; 1.0549x over previous
import functools

import jax
import jax.numpy as jnp
from jax import lax
from jax.experimental import pallas as pl
from jax.experimental.pallas import tpu as pltpu
from jax.experimental.pallas import tpu_sc as plsc

NUM_EXPERT = 8
D_MODEL = 768
D_FF = 4 * D_MODEL
TOP_K = 2
BATCH = 4096
N_TOKENS = BATCH // TOP_K

M = 256
T = BATCH // M + NUM_EXPERT
NSORT = T * M

NC = 2
NS = 16
NW = NC * NS

_G_PER_W = NSORT // NW
_G_CHUNK = _G_PER_W // 4
_C_PER_W = N_TOKENS // NW
_LANES = 16


@functools.cache
def _get_sc_gather():
    mesh = plsc.VectorSubcoreMesh(core_axis_name="c", subcore_axis_name="s")
    nbuf = 3
    nch = _G_PER_W // _G_CHUNK

    @functools.partial(
        pl.kernel,
        out_type=jax.ShapeDtypeStruct((NSORT, D_MODEL), jnp.float32),
        mesh=mesh,
        scratch_types=(
            [pltpu.VMEM((nch, _G_CHUNK), jnp.int32)]
            + [pltpu.VMEM((_G_CHUNK, D_MODEL), jnp.float32)] * nbuf
            + [pltpu.SemaphoreType.DMA] * (2 * nbuf)
        ),
    )
    def _sc_gather_k(inp_hbm, idx_hbm, out_hbm, idx_v, b0, b1, b2,
                     g0, g1, g2, s0, s1, s2):
        bufs = [b0, b1, b2]
        gsems = [g0, g1, g2]
        ssems = [s0, s1, s2]
        wid = lax.axis_index("s") * NC + lax.axis_index("c")
        base = wid * _G_PER_W
        pltpu.sync_copy(idx_hbm.at[wid], idx_v)
        hg = [
            pltpu.async_copy(inp_hbm.at[idx_v.at[c]], bufs[c], gsems[c])
            for c in range(nbuf)
        ]
        hs = [None] * nbuf
        for c in range(nch):
            b = c % nbuf
            if c >= nbuf:
                hs[b].wait()
                hg.append(pltpu.async_copy(
                    inp_hbm.at[idx_v.at[c]], bufs[b], gsems[b]))
            hg[c].wait()
            hs[b] = pltpu.async_copy(
                bufs[b], out_hbm.at[pl.ds(base + c * _G_CHUNK, _G_CHUNK)],
                ssems[b])
        for c in range(max(0, nch - nbuf), nch):
            b = c % nbuf
            if hs[b] is not None:
                hs[b].wait()
                hs[b] = None

    return _sc_gather_k


def _sc_gather(inp, srow):
    return _get_sc_gather()(inp, srow)


@functools.cache
def _get_sc_combine():
    mesh = plsc.VectorSubcoreMesh(core_axis_name="c", subcore_axis_name="s")
    tok_c = _C_PER_W // 2

    @functools.partial(
        pl.kernel,
        out_type=jax.ShapeDtypeStruct((N_TOKENS, D_MODEL), jnp.float32),
        mesh=mesh,
        scratch_types=[
            pltpu.VMEM((2, 2 * tok_c), jnp.int32),
            pltpu.VMEM((2 * tok_c, D_MODEL), jnp.float32),
            pltpu.VMEM((2 * tok_c, D_MODEL), jnp.float32),
            pltpu.SemaphoreType.DMA,
            pltpu.SemaphoreType.DMA,
            pltpu.SemaphoreType.DMA,
            pltpu.SemaphoreType.DMA,
        ],
    )
    def _sc_combine_k(y_hbm, pos_hbm, out_hbm, idx_v, bA, bB, gA, gB, sA, sB):
        wid = lax.axis_index("s") * NC + lax.axis_index("c")
        tbase = wid * _C_PER_W
        pltpu.sync_copy(pos_hbm.at[wid], idx_v)
        bufs = [bA, bB]
        hg = [pltpu.async_copy(y_hbm.at[idx_v.at[c]], bufs[c], [gA, gB][c])
              for c in range(2)]
        hs = []
        for c in range(2):
            buf = bufs[c]
            hg[c].wait()

            def body(i, carry, buf=buf):
                for k in range(D_MODEL // _LANES):
                    sl = pl.ds(k * _LANES, _LANES)
                    buf[i, sl] = buf[2 * i, sl] + buf[2 * i + 1, sl]
                return carry

            lax.fori_loop(0, tok_c, body, 0)
            hs.append(pltpu.async_copy(
                buf.at[pl.ds(0, tok_c)],
                out_hbm.at[pl.ds(tbase + c * tok_c, tok_c)], [sA, sB][c]))
        for h in hs:
            h.wait()

    return _sc_combine_k


def _sc_combine(y_scaled, pos):
    return _get_sc_combine()(y_scaled, pos)


def _ffn_kernel(eid_ref, flag_ref,
                x_ref, w1_ref, w2_ref, score_ref,
                y_ref):
    t = pl.program_id(0)

    @pl.when(flag_ref[t] == 1)
    def _body():
        h = lax.dot_general(
            x_ref[...], w1_ref[0],
            (((1,), (1,)), ((), ())), preferred_element_type=jnp.float32)
        h = jax.nn.gelu(h, approximate=True)
        y = lax.dot_general(
            h, w2_ref[0],
            (((1,), (1,)), ((), ())), preferred_element_type=jnp.float32)
        y_ref[...] = y * score_ref[...]


def _ffn(x_sorted, tile_eid, tile_flag, score_sorted, w1, w2):
    grid_spec = pltpu.PrefetchScalarGridSpec(
        num_scalar_prefetch=2,
        grid=(T,),
        in_specs=[
            pl.BlockSpec((M, D_MODEL), lambda t, e, f: (t, 0)),
            pl.BlockSpec((1, D_FF, D_MODEL), lambda t, e, f: (e[t], 0, 0)),
            pl.BlockSpec((1, D_MODEL, D_FF), lambda t, e, f: (e[t], 0, 0)),
            pl.BlockSpec((M, 1), lambda t, e, f: (t, 0)),
        ],
        out_specs=pl.BlockSpec((M, D_MODEL), lambda t, e, f: (t, 0)),
    )
    return pl.pallas_call(
        _ffn_kernel,
        grid_spec=grid_spec,
        out_shape=jax.ShapeDtypeStruct((NSORT, D_MODEL), jnp.float32),
    )(tile_eid, tile_flag, x_sorted, w1, w2, score_sorted)


def kernel(inp, gate_idx, gate_score, weight_htoh4, weight_h4toh):
    g = gate_idx.astype(jnp.int32)
    onehot = (g[:, None] == jnp.arange(NUM_EXPERT)[None, :]).astype(jnp.int32)
    incl = jnp.cumsum(onehot, axis=0)
    rank = jnp.sum((incl - onehot) * onehot, axis=1)
    counts = incl[-1]
    tiles_e = (counts + M - 1) // M
    tstart = jnp.concatenate(
        [jnp.zeros((1,), jnp.int32), jnp.cumsum(tiles_e)[:-1].astype(jnp.int32)])

    t = jnp.arange(T, dtype=jnp.int32)
    belongs = (t[:, None] >= tstart[None, :]) & (
        t[:, None] < (tstart + tiles_e)[None, :])
    has_e = belongs.any(axis=1)
    raw_eid = jnp.where(has_e, jnp.argmax(belongs, axis=1), 0).astype(jnp.int32)
    tile_eid = lax.cummax(raw_eid)
    tile_flag = has_e.astype(jnp.int32)

    pos = (tstart[g] * M + rank).astype(jnp.int32)
    pad_rows = jnp.arange(NSORT, dtype=jnp.int32) % BATCH
    srow = pad_rows.at[pos].set(jnp.arange(BATCH, dtype=jnp.int32))
    score_sorted = jnp.zeros((NSORT,), jnp.float32).at[pos].set(
        gate_score.reshape(-1)).reshape(NSORT, 1)

    x_sorted = _sc_gather(inp, srow.reshape(NW, -1, _G_CHUNK))
    y_scaled = _ffn(x_sorted, tile_eid, tile_flag, score_sorted,
                    weight_htoh4, weight_h4toh)
    return _sc_combine(y_scaled, pos.reshape(NW, 2, -1))
